# baseline (speedup 1.0000x reference)
import jax
import jax.numpy as jnp
from jax.experimental import pallas as pl
from jax.experimental.pallas import tpu as pltpu
from jax.experimental.pallas import tpu_sc as plsc

_B, _T, _N, _F = 16, 32, 68, 128
_SD, _TD, _NC, _E = 256, 256, 500, 680
_CD = 256
_TB = _T * _B
_FPP = 2
_K = _FPP * _B

_DNT = (((1,), (1,)), ((), ()))


def _dot(a, b, precision=None):
    return jnp.dot(a, b, preferred_element_type=jnp.float32, precision=precision)

_FAST = jax.lax.Precision.DEFAULT


def _dot_t(a, b):
    return jax.lax.dot_general(a, b, _DNT, preferred_element_type=jnp.float32)


_NSC, _NSUB = 2, 16
_EPW = 32
_ESLOTS = _NSC * _NSUB * _EPW
_ABUF = _N * _N
_ABUFP = 8192
_ZCH = _ABUFP // _NSUB


def _count_body(i1_ref, i2_ref, out_ref, idx1_v, idx2_v, ones_v, zv, cbuf, s1, s2):
    cid = jax.lax.axis_index("c")
    sid = jax.lax.axis_index("s")
    wid = cid * _NSUB + sid

    @pl.loop(0, _EPW, step=16)
    def _(i):
        ones_v[pl.ds(i, 16)] = jnp.ones((16,), jnp.float32)

    @pl.loop(0, _ZCH, step=16)
    def _(i):
        zv[pl.ds(i, 16)] = jnp.zeros((16,), jnp.float32)

    sl = pl.ds(sid * _ZCH, _ZCH)
    pltpu.sync_copy(zv, s1.at[sl])
    pltpu.sync_copy(zv, s2.at[sl])
    plsc.subcore_barrier()

    pltpu.sync_copy(i1_ref.at[wid], idx1_v)
    pltpu.sync_copy(i2_ref.at[wid], idx2_v)
    pltpu.sync_copy(ones_v, s1.at[idx1_v.at[0]], add=True)
    pltpu.sync_copy(ones_v, s2.at[idx2_v.at[0]], add=True)
    plsc.subcore_barrier()

    obase = cid * 2 * _ABUFP + sid * _ZCH
    pltpu.sync_copy(s1.at[sl], cbuf)
    pltpu.sync_copy(cbuf, out_ref.at[pl.ds(obase, _ZCH)])
    pltpu.sync_copy(s2.at[sl], cbuf)
    pltpu.sync_copy(cbuf, out_ref.at[pl.ds(obase + _ABUFP, _ZCH)])


def _sc_edge_counts(i1, i2):
    return pl.kernel(
        _count_body,
        out_type=jax.ShapeDtypeStruct((_NSC * 2 * _ABUFP,), jnp.float32),
        mesh=plsc.VectorSubcoreMesh(core_axis_name="c", subcore_axis_name="s"),
        scratch_types=[
            pltpu.VMEM((1, _EPW), jnp.int32),
            pltpu.VMEM((1, _EPW), jnp.int32),
            pltpu.VMEM((_EPW,), jnp.float32),
            pltpu.VMEM((_ZCH,), jnp.float32),
            pltpu.VMEM((_ZCH,), jnp.float32),
            pltpu.VMEM_SHARED((_ABUFP,), jnp.float32),
            pltpu.VMEM_SHARED((_ABUFP,), jnp.float32),
        ],
    )(i1, i2)


def _adj_norm_body(c_ref, ct_ref, a_ref):
    count = c_ref[...]
    deg_c = jnp.sum(count, axis=1, keepdims=True) + 1.0
    deg_r = jnp.sum(ct_ref[...], axis=0, keepdims=True) + 1.0
    eye = (
        jax.lax.broadcasted_iota(jnp.int32, (_N, _N), 0)
        == jax.lax.broadcasted_iota(jnp.int32, (_N, _N), 1)
    ).astype(jnp.float32)
    a_ref[...] = (count + eye) * jax.lax.rsqrt(deg_c) * jax.lax.rsqrt(deg_r)


def _gcn_body(a_ref, w0_ref, b0_ref, w1_ref, b1_ref, w2_ref, b2_ref, x_ref, out_ref):
    A = a_ref[...]

    def layer(h, w_ref, b_ref, bt, din):
        w = w_ref[...]
        y = jnp.concatenate(
            [_dot(h[:, i * din : (i + 1) * din], w) for i in range(_K)], axis=1
        )
        return jnp.maximum(_dot(A, y) + bt, 0.0)

    h = jnp.concatenate(
        [x_ref[i, j] for j in range(_FPP) for i in range(_B)], axis=1
    )
    bts = [jnp.tile(b_ref[...], (1, _K)) for b_ref in (b0_ref, b1_ref, b2_ref)]
    h = layer(h, w0_ref, b0_ref, bts[0], _F)
    h = layer(h, w1_ref, b1_ref, bts[1], _SD)
    h = layer(h, w2_ref, b2_ref, bts[2], _SD)
    out_ref[0, 0, :] = jnp.mean(h, axis=0)


def _lstm_body(
    seq_ref,
    wih0f_ref,
    whh0f_ref,
    wih0b_ref,
    whh0b_ref,
    wih1f_ref,
    whh1f_ref,
    wih1b_ref,
    whh1b_ref,
    bias0_ref,
    bias1_ref,
    clsw1_ref,
    clsb1_ref,
    clsw2_ref,
    clsb2_ref,
    out_ref,
    g0_ref,
    seq1_ref,
    g1_ref,
):
    H4 = 4 * _TD

    seq = seq_ref[...]
    g0_ref[:, 0:H4] = _dot_t(seq, wih0f_ref[...]) + bias0_ref[:, 0:H4]
    g0_ref[:, H4 : 2 * H4] = _dot_t(seq, wih0b_ref[...]) + bias0_ref[:, H4 : 2 * H4]

    def cell(g, c):
        i = jax.nn.sigmoid(g[:, 0:_TD])
        f = jax.nn.sigmoid(g[:, _TD : 2 * _TD])
        gg = jnp.tanh(g[:, 2 * _TD : 3 * _TD])
        o = jax.nn.sigmoid(g[:, 3 * _TD : 4 * _TD])
        c = f * c + i * gg
        return o * jnp.tanh(c), c

    def step0(s, carry):
        hf, cf, hb, cb = carry
        gf = g0_ref[pl.ds(s * _B, _B), 0:H4] + _dot_t(hf, whh0f_ref[...])
        gb = g0_ref[pl.ds((_T - 1 - s) * _B, _B), H4 : 2 * H4] + _dot_t(
            hb, whh0b_ref[...]
        )
        hf, cf = cell(gf, cf)
        hb, cb = cell(gb, cb)
        seq1_ref[pl.ds(s * _B, _B), 0:_TD] = hf
        seq1_ref[pl.ds((_T - 1 - s) * _B, _B), _TD : 2 * _TD] = hb
        return hf, cf, hb, cb

    z = jnp.zeros((_B, _TD), jnp.float32)
    jax.lax.fori_loop(0, _T, step0, (z, z, z, z))

    seq1 = seq1_ref[...]
    g1_ref[:, 0:H4] = _dot_t(seq1, wih1f_ref[...]) + bias1_ref[:, 0:H4]
    g1_ref[:, H4 : 2 * H4] = _dot_t(seq1, wih1b_ref[...]) + bias1_ref[:, H4 : 2 * H4]

    def step1(s, carry):
        hf, cf, hb, cb = carry
        gf = g1_ref[pl.ds(s * _B, _B), 0:H4] + _dot_t(hf, whh1f_ref[...])
        gb = g1_ref[pl.ds((_T - 1 - s) * _B, _B), H4 : 2 * H4] + _dot_t(
            hb, whh1b_ref[...]
        )
        hf, cf = cell(gf, cf)
        hb, cb = cell(gb, cb)
        return hf, cf, hb, cb

    h1f, _, h1b, _ = jax.lax.fori_loop(0, _T, step1, (z, z, z, z))

    to = jnp.concatenate([h1f, h1b], axis=1)
    h = jnp.maximum(_dot(to, clsw1_ref[...]) + clsb1_ref[...], 0.0)
    out_ref[...] = _dot(h, clsw2_ref[...]) + clsb2_ref[...]


@jax.jit
def kernel(x_temporal, edge_index, gcn_W0, gcn_b0, gcn_W1, gcn_b1, gcn_W2, gcn_b2, lstm_Wih_l0f, lstm_Whh_l0f, lstm_bih_l0f, lstm_bhh_l0f, lstm_Wih_l0b, lstm_Whh_l0b, lstm_bih_l0b, lstm_bhh_l0b, lstm_Wih_l1f, lstm_Whh_l1f, lstm_bih_l1f, lstm_bhh_l1f, lstm_Wih_l1b, lstm_Whh_l1b, lstm_bih_l1b, lstm_bhh_l1b, cls_W1, cls_b1, bn_gamma, bn_beta, bn_mean, bn_var, cls_W2, cls_b2):
    f32 = jnp.float32
    H4 = 4 * _TD

    srcn = edge_index[0]
    dstn = edge_index[1]
    pad = _ABUF + (jnp.arange(_ESLOTS - _E, dtype=jnp.int32) % (_ABUFP - _ABUF))
    i1 = jnp.concatenate([dstn * _N + srcn, pad]).reshape(_NSC * _NSUB, 1, _EPW)
    i2 = jnp.concatenate([srcn * _N + dstn, pad]).reshape(_NSC * _NSUB, 1, _EPW)
    parts = _sc_edge_counts(i1, i2).reshape(_NSC, 2, _ABUFP)
    csum = parts[0] + parts[1]
    c1 = csum[0, :_ABUF].reshape(_N, _N)
    c1T = csum[1, :_ABUF].reshape(_N, _N)
    A = pl.pallas_call(
        _adj_norm_body,
        out_shape=jax.ShapeDtypeStruct((_N, _N), f32),
    )(c1, c1T)

    bt = [b.reshape(1, _SD) for b in (gcn_b0, gcn_b1, gcn_b2)]
    ngp = _T // _FPP
    seq = pl.pallas_call(
        _gcn_body,
        grid=(ngp,),
        in_specs=[
            pl.BlockSpec((_N, _N), lambda t: (0, 0)),
            pl.BlockSpec((_F, _SD), lambda t: (0, 0)),
            pl.BlockSpec((1, _SD), lambda t: (0, 0)),
            pl.BlockSpec((_SD, _SD), lambda t: (0, 0)),
            pl.BlockSpec((1, _SD), lambda t: (0, 0)),
            pl.BlockSpec((_SD, _SD), lambda t: (0, 0)),
            pl.BlockSpec((1, _SD), lambda t: (0, 0)),
            pl.BlockSpec((_B, _FPP, _N, _F), lambda t: (0, t, 0, 0)),
        ],
        out_specs=pl.BlockSpec((1, 1, _K * _SD), lambda t: (t, 0, 0)),
        out_shape=jax.ShapeDtypeStruct((ngp, 1, _K * _SD), f32),
    )(A, gcn_W0, bt[0], gcn_W1, bt[1], gcn_W2, bt[2], x_temporal)
    seq = seq.reshape(_TB, _SD)

    bias0 = jnp.concatenate(
        [lstm_bih_l0f + lstm_bhh_l0f, lstm_bih_l0b + lstm_bhh_l0b]
    ).reshape(1, 2 * H4)
    bias1 = jnp.concatenate(
        [lstm_bih_l1f + lstm_bhh_l1f, lstm_bih_l1b + lstm_bhh_l1b]
    ).reshape(1, 2 * H4)

    scale = bn_gamma * jax.lax.rsqrt(bn_var + 1e-5)
    w1s = cls_W1 * scale[None, :]
    b1s = ((cls_b1 - bn_mean) * scale + bn_beta).reshape(1, _CD)

    logits = pl.pallas_call(
        _lstm_body,
        out_shape=jax.ShapeDtypeStruct((_B, _NC), f32),
        scratch_shapes=[
            pltpu.VMEM((_TB, 2 * H4), f32),
            pltpu.VMEM((_TB, 2 * _TD), f32),
            pltpu.VMEM((_TB, 2 * H4), f32),
        ],
    )(
        seq,
        lstm_Wih_l0f,
        lstm_Whh_l0f,
        lstm_Wih_l0b,
        lstm_Whh_l0b,
        lstm_Wih_l1f,
        lstm_Whh_l1f,
        lstm_Wih_l1b,
        lstm_Whh_l1b,
        bias0,
        bias1,
        w1s,
        b1s,
        cls_W2,
        cls_b2.reshape(1, _NC),
    )
    return logits


# speedup vs baseline: 1.0045x; 1.0045x over previous
import jax
import jax.numpy as jnp
from jax.experimental import pallas as pl
from jax.experimental.pallas import tpu as pltpu
from jax.experimental.pallas import tpu_sc as plsc

_B, _T, _N, _F = 16, 32, 68, 128
_SD, _TD, _NC, _E = 256, 256, 500, 680
_CD = 256
_TB = _T * _B
_FPP = 2
_K = _FPP * _B

_DNT = (((1,), (1,)), ((), ()))


def _dot(a, b, precision=None):
    return jnp.dot(a, b, preferred_element_type=jnp.float32, precision=precision)

_FAST = jax.lax.Precision.DEFAULT


def _dot_t(a, b):
    return jax.lax.dot_general(a, b, _DNT, preferred_element_type=jnp.float32)


_NSC, _NSUB = 2, 16
_EPW = 32
_ESLOTS = _NSC * _NSUB * _EPW
_ABUF = _N * _N
_ABUFP = 8192
_SBUF = 2 * _ABUFP
_ZCH = _SBUF // _NSUB


def _count_body(i12_ref, out_ref, idx_v, ones_v, zv, cbuf, s):
    cid = jax.lax.axis_index("c")
    sid = jax.lax.axis_index("s")
    wid = cid * _NSUB + sid

    @pl.loop(0, 2 * _EPW, step=16)
    def _(i):
        ones_v[pl.ds(i, 16)] = jnp.ones((16,), jnp.float32)

    @pl.loop(0, _ZCH, step=16)
    def _(i):
        zv[pl.ds(i, 16)] = jnp.zeros((16,), jnp.float32)

    sl = pl.ds(sid * _ZCH, _ZCH)
    pltpu.sync_copy(zv, s.at[sl])
    plsc.subcore_barrier()

    pltpu.sync_copy(i12_ref.at[wid], idx_v)
    pltpu.sync_copy(ones_v, s.at[idx_v.at[0]], add=True)
    plsc.subcore_barrier()

    pltpu.sync_copy(s.at[sl], cbuf)
    pltpu.sync_copy(cbuf, out_ref.at[pl.ds(cid * _SBUF + sid * _ZCH, _ZCH)])


def _sc_edge_counts(i12):
    return pl.kernel(
        _count_body,
        out_type=jax.ShapeDtypeStruct((_NSC * _SBUF,), jnp.float32),
        mesh=plsc.VectorSubcoreMesh(core_axis_name="c", subcore_axis_name="s"),
        scratch_types=[
            pltpu.VMEM((1, 2 * _EPW), jnp.int32),
            pltpu.VMEM((2 * _EPW,), jnp.float32),
            pltpu.VMEM((_ZCH,), jnp.float32),
            pltpu.VMEM((_ZCH,), jnp.float32),
            pltpu.VMEM_SHARED((_SBUF,), jnp.float32),
        ],
    )(i12)


def _adj_norm_body(c_ref, ct_ref, a_ref):
    count = c_ref[...]
    deg_c = jnp.sum(count, axis=1, keepdims=True) + 1.0
    deg_r = jnp.sum(ct_ref[...], axis=0, keepdims=True) + 1.0
    eye = (
        jax.lax.broadcasted_iota(jnp.int32, (_N, _N), 0)
        == jax.lax.broadcasted_iota(jnp.int32, (_N, _N), 1)
    ).astype(jnp.float32)
    a_ref[...] = (count + eye) * jax.lax.rsqrt(deg_c) * jax.lax.rsqrt(deg_r)


def _gcn_body(a_ref, w0_ref, b0_ref, w1_ref, b1_ref, w2_ref, b2_ref, x_ref, out_ref):
    A = a_ref[...]

    def layer(h, w_ref, b_ref, bt, din):
        w = w_ref[...]
        y = jnp.concatenate(
            [_dot(h[:, i * din : (i + 1) * din], w) for i in range(_K)], axis=1
        )
        return jnp.maximum(_dot(A, y) + bt, 0.0)

    h = jnp.concatenate(
        [x_ref[i, j] for j in range(_FPP) for i in range(_B)], axis=1
    )
    bts = [jnp.tile(b_ref[...], (1, _K)) for b_ref in (b0_ref, b1_ref, b2_ref)]
    h = layer(h, w0_ref, b0_ref, bts[0], _F)
    h = layer(h, w1_ref, b1_ref, bts[1], _SD)
    h = layer(h, w2_ref, b2_ref, bts[2], _SD)
    out_ref[0, 0, :] = jnp.mean(h, axis=0)


def _lstm_body(
    seq_ref,
    wih0f_ref,
    whh0f_ref,
    wih0b_ref,
    whh0b_ref,
    wih1f_ref,
    whh1f_ref,
    wih1b_ref,
    whh1b_ref,
    bias0_ref,
    bias1_ref,
    clsw1_ref,
    clsb1_ref,
    clsw2_ref,
    clsb2_ref,
    out_ref,
    g0_ref,
    seq1_ref,
    g1_ref,
):
    H4 = 4 * _TD

    seq = seq_ref[...]
    g0_ref[:, 0:H4] = _dot_t(seq, wih0f_ref[...]) + bias0_ref[:, 0:H4]
    g0_ref[:, H4 : 2 * H4] = _dot_t(seq, wih0b_ref[...]) + bias0_ref[:, H4 : 2 * H4]

    def cell(g, c):
        i = jax.nn.sigmoid(g[:, 0:_TD])
        f = jax.nn.sigmoid(g[:, _TD : 2 * _TD])
        gg = jnp.tanh(g[:, 2 * _TD : 3 * _TD])
        o = jax.nn.sigmoid(g[:, 3 * _TD : 4 * _TD])
        c = f * c + i * gg
        return o * jnp.tanh(c), c

    def step0(s, carry):
        hf, cf, hb, cb = carry
        gf = g0_ref[pl.ds(s * _B, _B), 0:H4] + _dot_t(hf, whh0f_ref[...])
        gb = g0_ref[pl.ds((_T - 1 - s) * _B, _B), H4 : 2 * H4] + _dot_t(
            hb, whh0b_ref[...]
        )
        hf, cf = cell(gf, cf)
        hb, cb = cell(gb, cb)
        seq1_ref[pl.ds(s * _B, _B), 0:_TD] = hf
        seq1_ref[pl.ds((_T - 1 - s) * _B, _B), _TD : 2 * _TD] = hb
        return hf, cf, hb, cb

    z = jnp.zeros((_B, _TD), jnp.float32)
    jax.lax.fori_loop(0, _T, step0, (z, z, z, z))

    seq1 = seq1_ref[...]
    g1_ref[:, 0:H4] = _dot_t(seq1, wih1f_ref[...]) + bias1_ref[:, 0:H4]
    g1_ref[:, H4 : 2 * H4] = _dot_t(seq1, wih1b_ref[...]) + bias1_ref[:, H4 : 2 * H4]

    def step1(s, carry):
        hf, cf, hb, cb = carry
        gf = g1_ref[pl.ds(s * _B, _B), 0:H4] + _dot_t(hf, whh1f_ref[...])
        gb = g1_ref[pl.ds((_T - 1 - s) * _B, _B), H4 : 2 * H4] + _dot_t(
            hb, whh1b_ref[...]
        )
        hf, cf = cell(gf, cf)
        hb, cb = cell(gb, cb)
        return hf, cf, hb, cb

    h1f, _, h1b, _ = jax.lax.fori_loop(0, _T, step1, (z, z, z, z))

    to = jnp.concatenate([h1f, h1b], axis=1)
    h = jnp.maximum(_dot(to, clsw1_ref[...]) + clsb1_ref[...], 0.0)
    out_ref[...] = _dot(h, clsw2_ref[...]) + clsb2_ref[...]


@jax.jit
def kernel(x_temporal, edge_index, gcn_W0, gcn_b0, gcn_W1, gcn_b1, gcn_W2, gcn_b2, lstm_Wih_l0f, lstm_Whh_l0f, lstm_bih_l0f, lstm_bhh_l0f, lstm_Wih_l0b, lstm_Whh_l0b, lstm_bih_l0b, lstm_bhh_l0b, lstm_Wih_l1f, lstm_Whh_l1f, lstm_bih_l1f, lstm_bhh_l1f, lstm_Wih_l1b, lstm_Whh_l1b, lstm_bih_l1b, lstm_bhh_l1b, cls_W1, cls_b1, bn_gamma, bn_beta, bn_mean, bn_var, cls_W2, cls_b2):
    f32 = jnp.float32
    H4 = 4 * _TD

    srcn = edge_index[0]
    dstn = edge_index[1]
    npad = _ESLOTS - _E
    pad = _ABUF + (jnp.arange(npad, dtype=jnp.int32) % (_ABUFP - _ABUF))
    f1 = jnp.concatenate([dstn * _N + srcn, pad]).reshape(_NSC * _NSUB, _EPW)
    f2 = _ABUFP + jnp.concatenate([srcn * _N + dstn, pad]).reshape(
        _NSC * _NSUB, _EPW
    )
    i12 = jnp.concatenate([f1, f2], axis=1).reshape(_NSC * _NSUB, 1, 2 * _EPW)
    parts = _sc_edge_counts(i12).reshape(_NSC, 2, _ABUFP)
    csum = parts[0] + parts[1]
    c1 = csum[0, :_ABUF].reshape(_N, _N)
    c1T = csum[1, :_ABUF].reshape(_N, _N)
    A = pl.pallas_call(
        _adj_norm_body,
        out_shape=jax.ShapeDtypeStruct((_N, _N), f32),
    )(c1, c1T)

    bt = [b.reshape(1, _SD) for b in (gcn_b0, gcn_b1, gcn_b2)]
    ngp = _T // _FPP
    seq = pl.pallas_call(
        _gcn_body,
        grid=(ngp,),
        in_specs=[
            pl.BlockSpec((_N, _N), lambda t: (0, 0)),
            pl.BlockSpec((_F, _SD), lambda t: (0, 0)),
            pl.BlockSpec((1, _SD), lambda t: (0, 0)),
            pl.BlockSpec((_SD, _SD), lambda t: (0, 0)),
            pl.BlockSpec((1, _SD), lambda t: (0, 0)),
            pl.BlockSpec((_SD, _SD), lambda t: (0, 0)),
            pl.BlockSpec((1, _SD), lambda t: (0, 0)),
            pl.BlockSpec((_B, _FPP, _N, _F), lambda t: (0, t, 0, 0)),
        ],
        out_specs=pl.BlockSpec((1, 1, _K * _SD), lambda t: (t, 0, 0)),
        out_shape=jax.ShapeDtypeStruct((ngp, 1, _K * _SD), f32),
    )(A, gcn_W0, bt[0], gcn_W1, bt[1], gcn_W2, bt[2], x_temporal)
    seq = seq.reshape(_TB, _SD)

    bias0 = jnp.concatenate(
        [lstm_bih_l0f + lstm_bhh_l0f, lstm_bih_l0b + lstm_bhh_l0b]
    ).reshape(1, 2 * H4)
    bias1 = jnp.concatenate(
        [lstm_bih_l1f + lstm_bhh_l1f, lstm_bih_l1b + lstm_bhh_l1b]
    ).reshape(1, 2 * H4)

    scale = bn_gamma * jax.lax.rsqrt(bn_var + 1e-5)
    w1s = cls_W1 * scale[None, :]
    b1s = ((cls_b1 - bn_mean) * scale + bn_beta).reshape(1, _CD)

    logits = pl.pallas_call(
        _lstm_body,
        out_shape=jax.ShapeDtypeStruct((_B, _NC), f32),
        scratch_shapes=[
            pltpu.VMEM((_TB, 2 * H4), f32),
            pltpu.VMEM((_TB, 2 * _TD), f32),
            pltpu.VMEM((_TB, 2 * H4), f32),
        ],
    )(
        seq,
        lstm_Wih_l0f,
        lstm_Whh_l0f,
        lstm_Wih_l0b,
        lstm_Whh_l0b,
        lstm_Wih_l1f,
        lstm_Whh_l1f,
        lstm_Wih_l1b,
        lstm_Whh_l1b,
        bias0,
        bias1,
        w1s,
        b1s,
        cls_W2,
        cls_b2.reshape(1, _NC),
    )
    return logits


# speedup vs baseline: 1.0222x; 1.0177x over previous
import jax
import jax.numpy as jnp
from jax.experimental import pallas as pl
from jax.experimental.pallas import tpu as pltpu
from jax.experimental.pallas import tpu_sc as plsc

_B, _T, _N, _F = 16, 32, 68, 128
_SD, _TD, _NC, _E = 256, 256, 500, 680
_CD = 256
_TB = _T * _B
_FPP = 2
_K = _FPP * _B

_DNT = (((1,), (1,)), ((), ()))


def _dot(a, b, precision=None):
    return jnp.dot(a, b, preferred_element_type=jnp.float32, precision=precision)

_FAST = jax.lax.Precision.DEFAULT


def _dot_t(a, b):
    return jax.lax.dot_general(a, b, _DNT, preferred_element_type=jnp.float32)


_NSC, _NSUB = 2, 16
_EPW = 32
_ESLOTS = _NSC * _NSUB * _EPW
_STR = 80
_ABUF = _N * _STR
_ABUFP = 8192
_SBUF = 2 * _ABUFP
_ZCH = _SBUF // _NSUB


def _count_body(i12_ref, out_ref, idx_v, ones_v, zv, cbuf, s):
    cid = jax.lax.axis_index("c")
    sid = jax.lax.axis_index("s")
    wid = cid * _NSUB + sid

    @pl.loop(0, 2 * _EPW, step=16)
    def _(i):
        ones_v[pl.ds(i, 16)] = jnp.ones((16,), jnp.float32)

    @pl.loop(0, _ZCH, step=16)
    def _(i):
        zv[pl.ds(i, 16)] = jnp.zeros((16,), jnp.float32)

    sl = pl.ds(sid * _ZCH, _ZCH)
    pltpu.sync_copy(zv, s.at[sl])
    plsc.subcore_barrier()

    pltpu.sync_copy(i12_ref.at[wid], idx_v)
    pltpu.sync_copy(ones_v, s.at[idx_v.at[0]], add=True)
    plsc.subcore_barrier()

    pltpu.sync_copy(s.at[sl], cbuf)
    pltpu.sync_copy(cbuf, out_ref.at[pl.ds(cid * _SBUF + sid * _ZCH, _ZCH)])


def _sc_edge_counts(i12):
    return pl.kernel(
        _count_body,
        out_type=jax.ShapeDtypeStruct((_NSC * _SBUF,), jnp.float32),
        mesh=plsc.VectorSubcoreMesh(core_axis_name="c", subcore_axis_name="s"),
        scratch_types=[
            pltpu.VMEM((1, 2 * _EPW), jnp.int32),
            pltpu.VMEM((2 * _EPW,), jnp.float32),
            pltpu.VMEM((_ZCH,), jnp.float32),
            pltpu.VMEM((_ZCH,), jnp.float32),
            pltpu.VMEM_SHARED((_SBUF,), jnp.float32),
        ],
    )(i12)


def _gcn_body(cs_ref, w0_ref, b0_ref, w1_ref, b1_ref, w2_ref, b2_ref, x_ref, out_ref):
    cnt = cs_ref[0, :, 0:_N]
    cntT = cs_ref[1, :, 0:_N]
    deg_c = jnp.sum(cnt, axis=1, keepdims=True) + 1.0
    deg_r = jnp.sum(cntT, axis=0, keepdims=True) + 1.0
    eye = (
        jax.lax.broadcasted_iota(jnp.int32, (_N, _N), 0)
        == jax.lax.broadcasted_iota(jnp.int32, (_N, _N), 1)
    ).astype(jnp.float32)
    A = (cnt + eye) * jax.lax.rsqrt(deg_c) * jax.lax.rsqrt(deg_r)

    def layer(h, w_ref, b_ref, bt, din):
        w = w_ref[...]
        y = jnp.concatenate(
            [_dot(h[:, i * din : (i + 1) * din], w) for i in range(_K)], axis=1
        )
        return jnp.maximum(_dot(A, y) + bt, 0.0)

    h = jnp.concatenate(
        [x_ref[i, j] for j in range(_FPP) for i in range(_B)], axis=1
    )
    bts = [jnp.tile(b_ref[...], (1, _K)) for b_ref in (b0_ref, b1_ref, b2_ref)]
    h = layer(h, w0_ref, b0_ref, bts[0], _F)
    h = layer(h, w1_ref, b1_ref, bts[1], _SD)
    h = layer(h, w2_ref, b2_ref, bts[2], _SD)
    out_ref[0, 0, :] = jnp.mean(h, axis=0)


def _lstm_body(
    seq_ref,
    wih0f_ref,
    whh0f_ref,
    wih0b_ref,
    whh0b_ref,
    wih1f_ref,
    whh1f_ref,
    wih1b_ref,
    whh1b_ref,
    bias0_ref,
    bias1_ref,
    clsw1_ref,
    clsb1_ref,
    clsw2_ref,
    clsb2_ref,
    out_ref,
    g0_ref,
    seq1_ref,
    g1_ref,
):
    H4 = 4 * _TD

    seq = seq_ref[...]
    g0_ref[:, 0:H4] = _dot_t(seq, wih0f_ref[...]) + bias0_ref[:, 0:H4]
    g0_ref[:, H4 : 2 * H4] = _dot_t(seq, wih0b_ref[...]) + bias0_ref[:, H4 : 2 * H4]

    def cell(g, c):
        i = jax.nn.sigmoid(g[:, 0:_TD])
        f = jax.nn.sigmoid(g[:, _TD : 2 * _TD])
        gg = jnp.tanh(g[:, 2 * _TD : 3 * _TD])
        o = jax.nn.sigmoid(g[:, 3 * _TD : 4 * _TD])
        c = f * c + i * gg
        return o * jnp.tanh(c), c

    def step0(s, carry):
        hf, cf, hb, cb = carry
        gf = g0_ref[pl.ds(s * _B, _B), 0:H4] + _dot_t(hf, whh0f_ref[...])
        gb = g0_ref[pl.ds((_T - 1 - s) * _B, _B), H4 : 2 * H4] + _dot_t(
            hb, whh0b_ref[...]
        )
        hf, cf = cell(gf, cf)
        hb, cb = cell(gb, cb)
        seq1_ref[pl.ds(s * _B, _B), 0:_TD] = hf
        seq1_ref[pl.ds((_T - 1 - s) * _B, _B), _TD : 2 * _TD] = hb
        return hf, cf, hb, cb

    z = jnp.zeros((_B, _TD), jnp.float32)
    jax.lax.fori_loop(0, _T, step0, (z, z, z, z))

    seq1 = seq1_ref[...]
    g1_ref[:, 0:H4] = _dot_t(seq1, wih1f_ref[...]) + bias1_ref[:, 0:H4]
    g1_ref[:, H4 : 2 * H4] = _dot_t(seq1, wih1b_ref[...]) + bias1_ref[:, H4 : 2 * H4]

    def step1(s, carry):
        hf, cf, hb, cb = carry
        gf = g1_ref[pl.ds(s * _B, _B), 0:H4] + _dot_t(hf, whh1f_ref[...])
        gb = g1_ref[pl.ds((_T - 1 - s) * _B, _B), H4 : 2 * H4] + _dot_t(
            hb, whh1b_ref[...]
        )
        hf, cf = cell(gf, cf)
        hb, cb = cell(gb, cb)
        return hf, cf, hb, cb

    h1f, _, h1b, _ = jax.lax.fori_loop(0, _T, step1, (z, z, z, z))

    to = jnp.concatenate([h1f, h1b], axis=1)
    h = jnp.maximum(_dot(to, clsw1_ref[...]) + clsb1_ref[...], 0.0)
    out_ref[...] = _dot(h, clsw2_ref[...]) + clsb2_ref[...]


@jax.jit
def kernel(x_temporal, edge_index, gcn_W0, gcn_b0, gcn_W1, gcn_b1, gcn_W2, gcn_b2, lstm_Wih_l0f, lstm_Whh_l0f, lstm_bih_l0f, lstm_bhh_l0f, lstm_Wih_l0b, lstm_Whh_l0b, lstm_bih_l0b, lstm_bhh_l0b, lstm_Wih_l1f, lstm_Whh_l1f, lstm_bih_l1f, lstm_bhh_l1f, lstm_Wih_l1b, lstm_Whh_l1b, lstm_bih_l1b, lstm_bhh_l1b, cls_W1, cls_b1, bn_gamma, bn_beta, bn_mean, bn_var, cls_W2, cls_b2):
    f32 = jnp.float32
    H4 = 4 * _TD

    srcn = edge_index[0]
    dstn = edge_index[1]
    npad = _ESLOTS - _E
    pad = _ABUF + (jnp.arange(npad, dtype=jnp.int32) % (_ABUFP - _ABUF))
    f1 = jnp.concatenate([dstn * _STR + srcn, pad]).reshape(_NSC * _NSUB, _EPW)
    f2 = _ABUFP + jnp.concatenate([srcn * _STR + dstn, pad]).reshape(
        _NSC * _NSUB, _EPW
    )
    i12 = jnp.concatenate([f1, f2], axis=1).reshape(_NSC * _NSUB, 1, 2 * _EPW)
    parts = _sc_edge_counts(i12).reshape(_NSC, 2, _ABUFP)
    csum = parts[0] + parts[1]
    cs2 = csum[:, :_ABUF].reshape(2, _N, _STR)

    bt = [b.reshape(1, _SD) for b in (gcn_b0, gcn_b1, gcn_b2)]
    ngp = _T // _FPP
    seq = pl.pallas_call(
        _gcn_body,
        grid=(ngp,),
        in_specs=[
            pl.BlockSpec((2, _N, _STR), lambda t: (0, 0, 0)),
            pl.BlockSpec((_F, _SD), lambda t: (0, 0)),
            pl.BlockSpec((1, _SD), lambda t: (0, 0)),
            pl.BlockSpec((_SD, _SD), lambda t: (0, 0)),
            pl.BlockSpec((1, _SD), lambda t: (0, 0)),
            pl.BlockSpec((_SD, _SD), lambda t: (0, 0)),
            pl.BlockSpec((1, _SD), lambda t: (0, 0)),
            pl.BlockSpec((_B, _FPP, _N, _F), lambda t: (0, t, 0, 0)),
        ],
        out_specs=pl.BlockSpec((1, 1, _K * _SD), lambda t: (t, 0, 0)),
        out_shape=jax.ShapeDtypeStruct((ngp, 1, _K * _SD), f32),
    )(cs2, gcn_W0, bt[0], gcn_W1, bt[1], gcn_W2, bt[2], x_temporal)
    seq = seq.reshape(_TB, _SD)

    bias0 = jnp.concatenate(
        [lstm_bih_l0f + lstm_bhh_l0f, lstm_bih_l0b + lstm_bhh_l0b]
    ).reshape(1, 2 * H4)
    bias1 = jnp.concatenate(
        [lstm_bih_l1f + lstm_bhh_l1f, lstm_bih_l1b + lstm_bhh_l1b]
    ).reshape(1, 2 * H4)

    scale = bn_gamma * jax.lax.rsqrt(bn_var + 1e-5)
    w1s = cls_W1 * scale[None, :]
    b1s = ((cls_b1 - bn_mean) * scale + bn_beta).reshape(1, _CD)

    logits = pl.pallas_call(
        _lstm_body,
        out_shape=jax.ShapeDtypeStruct((_B, _NC), f32),
        scratch_shapes=[
            pltpu.VMEM((_TB, 2 * H4), f32),
            pltpu.VMEM((_TB, 2 * _TD), f32),
            pltpu.VMEM((_TB, 2 * H4), f32),
        ],
    )(
        seq,
        lstm_Wih_l0f,
        lstm_Whh_l0f,
        lstm_Wih_l0b,
        lstm_Whh_l0b,
        lstm_Wih_l1f,
        lstm_Whh_l1f,
        lstm_Wih_l1b,
        lstm_Whh_l1b,
        bias0,
        bias1,
        w1s,
        b1s,
        cls_W2,
        cls_b2.reshape(1, _NC),
    )
    return logits


# speedup vs baseline: 1.0376x; 1.0151x over previous
import jax
import jax.numpy as jnp
from jax.experimental import pallas as pl
from jax.experimental.pallas import tpu as pltpu
from jax.experimental.pallas import tpu_sc as plsc

_B, _T, _N, _F = 16, 32, 68, 128
_SD, _TD, _NC, _E = 256, 256, 500, 680
_CD = 256
_TB = _T * _B
_FPP = 4
_K = _FPP * _B

_DNT = (((1,), (1,)), ((), ()))


def _dot(a, b, precision=None):
    return jnp.dot(a, b, preferred_element_type=jnp.float32, precision=precision)

_FAST = jax.lax.Precision.DEFAULT


def _dot_t(a, b):
    return jax.lax.dot_general(a, b, _DNT, preferred_element_type=jnp.float32)


_NSC, _NSUB = 2, 16
_EPW = 32
_ESLOTS = _NSC * _NSUB * _EPW
_STR = 80
_ABUF = _N * _STR
_ABUFP = 8192
_SBUF = 2 * _ABUFP
_ZCH = _SBUF // _NSUB


def _count_body(i12_ref, out_ref, idx_v, ones_v, zv, cbuf, s):
    cid = jax.lax.axis_index("c")
    sid = jax.lax.axis_index("s")
    wid = cid * _NSUB + sid

    @pl.loop(0, 2 * _EPW, step=16)
    def _(i):
        ones_v[pl.ds(i, 16)] = jnp.ones((16,), jnp.float32)

    @pl.loop(0, _ZCH, step=16)
    def _(i):
        zv[pl.ds(i, 16)] = jnp.zeros((16,), jnp.float32)

    sl = pl.ds(sid * _ZCH, _ZCH)
    pltpu.sync_copy(zv, s.at[sl])
    plsc.subcore_barrier()

    pltpu.sync_copy(i12_ref.at[wid], idx_v)
    pltpu.sync_copy(ones_v, s.at[idx_v.at[0]], add=True)
    plsc.subcore_barrier()

    pltpu.sync_copy(s.at[sl], cbuf)
    pltpu.sync_copy(cbuf, out_ref.at[pl.ds(cid * _SBUF + sid * _ZCH, _ZCH)])


def _sc_edge_counts(i12):
    return pl.kernel(
        _count_body,
        out_type=jax.ShapeDtypeStruct((_NSC * _SBUF,), jnp.float32),
        mesh=plsc.VectorSubcoreMesh(core_axis_name="c", subcore_axis_name="s"),
        scratch_types=[
            pltpu.VMEM((1, 2 * _EPW), jnp.int32),
            pltpu.VMEM((2 * _EPW,), jnp.float32),
            pltpu.VMEM((_ZCH,), jnp.float32),
            pltpu.VMEM((_ZCH,), jnp.float32),
            pltpu.VMEM_SHARED((_SBUF,), jnp.float32),
        ],
    )(i12)


def _gcn_body(cs_ref, w0_ref, b0_ref, w1_ref, b1_ref, w2_ref, b2_ref, x_ref, out_ref):
    cnt = cs_ref[0, :, 0:_N]
    cntT = cs_ref[1, :, 0:_N]
    deg_c = jnp.sum(cnt, axis=1, keepdims=True) + 1.0
    deg_r = jnp.sum(cntT, axis=0, keepdims=True) + 1.0
    eye = (
        jax.lax.broadcasted_iota(jnp.int32, (_N, _N), 0)
        == jax.lax.broadcasted_iota(jnp.int32, (_N, _N), 1)
    ).astype(jnp.float32)
    A = (cnt + eye) * jax.lax.rsqrt(deg_c) * jax.lax.rsqrt(deg_r)

    def layer(h, w_ref, b_ref, bt, din):
        w = w_ref[...]
        y = jnp.concatenate(
            [_dot(h[:, i * din : (i + 1) * din], w) for i in range(_K)], axis=1
        )
        return jnp.maximum(_dot(A, y) + bt, 0.0)

    h = jnp.concatenate(
        [x_ref[i, j] for j in range(_FPP) for i in range(_B)], axis=1
    )
    bts = [jnp.tile(b_ref[...], (1, _K)) for b_ref in (b0_ref, b1_ref, b2_ref)]
    h = layer(h, w0_ref, b0_ref, bts[0], _F)
    h = layer(h, w1_ref, b1_ref, bts[1], _SD)
    h = layer(h, w2_ref, b2_ref, bts[2], _SD)
    out_ref[0, 0, :] = jnp.mean(h, axis=0)


def _lstm_body(
    seq_ref,
    wih0f_ref,
    whh0f_ref,
    wih0b_ref,
    whh0b_ref,
    wih1f_ref,
    whh1f_ref,
    wih1b_ref,
    whh1b_ref,
    bias0_ref,
    bias1_ref,
    clsw1_ref,
    clsb1_ref,
    clsw2_ref,
    clsb2_ref,
    out_ref,
    g0_ref,
    seq1_ref,
    g1_ref,
):
    H4 = 4 * _TD

    seq = seq_ref[...]
    g0_ref[:, 0:H4] = _dot_t(seq, wih0f_ref[...]) + bias0_ref[:, 0:H4]
    g0_ref[:, H4 : 2 * H4] = _dot_t(seq, wih0b_ref[...]) + bias0_ref[:, H4 : 2 * H4]

    def cell(g, c):
        i = jax.nn.sigmoid(g[:, 0:_TD])
        f = jax.nn.sigmoid(g[:, _TD : 2 * _TD])
        gg = jnp.tanh(g[:, 2 * _TD : 3 * _TD])
        o = jax.nn.sigmoid(g[:, 3 * _TD : 4 * _TD])
        c = f * c + i * gg
        return o * jnp.tanh(c), c

    def step0(s, carry):
        hf, cf, hb, cb = carry
        gf = g0_ref[pl.ds(s * _B, _B), 0:H4] + _dot_t(hf, whh0f_ref[...])
        gb = g0_ref[pl.ds((_T - 1 - s) * _B, _B), H4 : 2 * H4] + _dot_t(
            hb, whh0b_ref[...]
        )
        hf, cf = cell(gf, cf)
        hb, cb = cell(gb, cb)
        seq1_ref[pl.ds(s * _B, _B), 0:_TD] = hf
        seq1_ref[pl.ds((_T - 1 - s) * _B, _B), _TD : 2 * _TD] = hb
        return hf, cf, hb, cb

    z = jnp.zeros((_B, _TD), jnp.float32)
    jax.lax.fori_loop(0, _T, step0, (z, z, z, z))

    seq1 = seq1_ref[...]
    g1_ref[:, 0:H4] = _dot_t(seq1, wih1f_ref[...]) + bias1_ref[:, 0:H4]
    g1_ref[:, H4 : 2 * H4] = _dot_t(seq1, wih1b_ref[...]) + bias1_ref[:, H4 : 2 * H4]

    def step1(s, carry):
        hf, cf, hb, cb = carry
        gf = g1_ref[pl.ds(s * _B, _B), 0:H4] + _dot_t(hf, whh1f_ref[...])
        gb = g1_ref[pl.ds((_T - 1 - s) * _B, _B), H4 : 2 * H4] + _dot_t(
            hb, whh1b_ref[...]
        )
        hf, cf = cell(gf, cf)
        hb, cb = cell(gb, cb)
        return hf, cf, hb, cb

    h1f, _, h1b, _ = jax.lax.fori_loop(0, _T, step1, (z, z, z, z))

    to = jnp.concatenate([h1f, h1b], axis=1)
    h = jnp.maximum(_dot(to, clsw1_ref[...]) + clsb1_ref[...], 0.0)
    out_ref[...] = _dot(h, clsw2_ref[...]) + clsb2_ref[...]


@jax.jit
def kernel(x_temporal, edge_index, gcn_W0, gcn_b0, gcn_W1, gcn_b1, gcn_W2, gcn_b2, lstm_Wih_l0f, lstm_Whh_l0f, lstm_bih_l0f, lstm_bhh_l0f, lstm_Wih_l0b, lstm_Whh_l0b, lstm_bih_l0b, lstm_bhh_l0b, lstm_Wih_l1f, lstm_Whh_l1f, lstm_bih_l1f, lstm_bhh_l1f, lstm_Wih_l1b, lstm_Whh_l1b, lstm_bih_l1b, lstm_bhh_l1b, cls_W1, cls_b1, bn_gamma, bn_beta, bn_mean, bn_var, cls_W2, cls_b2):
    f32 = jnp.float32
    H4 = 4 * _TD

    srcn = edge_index[0]
    dstn = edge_index[1]
    npad = _ESLOTS - _E
    pad = _ABUF + (jnp.arange(npad, dtype=jnp.int32) % (_ABUFP - _ABUF))
    f1 = jnp.concatenate([dstn * _STR + srcn, pad]).reshape(_NSC * _NSUB, _EPW)
    f2 = _ABUFP + jnp.concatenate([srcn * _STR + dstn, pad]).reshape(
        _NSC * _NSUB, _EPW
    )
    i12 = jnp.concatenate([f1, f2], axis=1).reshape(_NSC * _NSUB, 1, 2 * _EPW)
    parts = _sc_edge_counts(i12).reshape(_NSC, 2, _ABUFP)
    csum = parts[0] + parts[1]
    cs2 = csum[:, :_ABUF].reshape(2, _N, _STR)

    bt = [b.reshape(1, _SD) for b in (gcn_b0, gcn_b1, gcn_b2)]
    ngp = _T // _FPP
    seq = pl.pallas_call(
        _gcn_body,
        grid=(ngp,),
        in_specs=[
            pl.BlockSpec((2, _N, _STR), lambda t: (0, 0, 0)),
            pl.BlockSpec((_F, _SD), lambda t: (0, 0)),
            pl.BlockSpec((1, _SD), lambda t: (0, 0)),
            pl.BlockSpec((_SD, _SD), lambda t: (0, 0)),
            pl.BlockSpec((1, _SD), lambda t: (0, 0)),
            pl.BlockSpec((_SD, _SD), lambda t: (0, 0)),
            pl.BlockSpec((1, _SD), lambda t: (0, 0)),
            pl.BlockSpec((_B, _FPP, _N, _F), lambda t: (0, t, 0, 0)),
        ],
        out_specs=pl.BlockSpec((1, 1, _K * _SD), lambda t: (t, 0, 0)),
        out_shape=jax.ShapeDtypeStruct((ngp, 1, _K * _SD), f32),
    )(cs2, gcn_W0, bt[0], gcn_W1, bt[1], gcn_W2, bt[2], x_temporal)
    seq = seq.reshape(_TB, _SD)

    bias0 = jnp.concatenate(
        [lstm_bih_l0f + lstm_bhh_l0f, lstm_bih_l0b + lstm_bhh_l0b]
    ).reshape(1, 2 * H4)
    bias1 = jnp.concatenate(
        [lstm_bih_l1f + lstm_bhh_l1f, lstm_bih_l1b + lstm_bhh_l1b]
    ).reshape(1, 2 * H4)

    scale = bn_gamma * jax.lax.rsqrt(bn_var + 1e-5)
    w1s = cls_W1 * scale[None, :]
    b1s = ((cls_b1 - bn_mean) * scale + bn_beta).reshape(1, _CD)

    logits = pl.pallas_call(
        _lstm_body,
        out_shape=jax.ShapeDtypeStruct((_B, _NC), f32),
        scratch_shapes=[
            pltpu.VMEM((_TB, 2 * H4), f32),
            pltpu.VMEM((_TB, 2 * _TD), f32),
            pltpu.VMEM((_TB, 2 * H4), f32),
        ],
    )(
        seq,
        lstm_Wih_l0f,
        lstm_Whh_l0f,
        lstm_Wih_l0b,
        lstm_Whh_l0b,
        lstm_Wih_l1f,
        lstm_Whh_l1f,
        lstm_Wih_l1b,
        lstm_Whh_l1b,
        bias0,
        bias1,
        w1s,
        b1s,
        cls_W2,
        cls_b2.reshape(1, _NC),
    )
    return logits


# speedup vs baseline: 1.0396x; 1.0019x over previous
import jax
import jax.numpy as jnp
from jax.experimental import pallas as pl
from jax.experimental.pallas import tpu as pltpu
from jax.experimental.pallas import tpu_sc as plsc

_B, _T, _N, _F = 16, 32, 68, 128
_SD, _TD, _NC, _E = 256, 256, 500, 680
_CD = 256
_TB = _T * _B
_FPP = 8
_K = _FPP * _B

_DNT = (((1,), (1,)), ((), ()))


def _dot(a, b, precision=None):
    return jnp.dot(a, b, preferred_element_type=jnp.float32, precision=precision)

_FAST = jax.lax.Precision.DEFAULT


def _dot_t(a, b):
    return jax.lax.dot_general(a, b, _DNT, preferred_element_type=jnp.float32)


_NSC, _NSUB = 2, 16
_EPW = 32
_ESLOTS = _NSC * _NSUB * _EPW
_STR = 80
_ABUF = _N * _STR
_ABUFP = 8192
_SBUF = 2 * _ABUFP
_ZCH = _SBUF // _NSUB


def _count_body(i12_ref, out_ref, idx_v, ones_v, zv, cbuf, s):
    cid = jax.lax.axis_index("c")
    sid = jax.lax.axis_index("s")
    wid = cid * _NSUB + sid

    @pl.loop(0, 2 * _EPW, step=16)
    def _(i):
        ones_v[pl.ds(i, 16)] = jnp.ones((16,), jnp.float32)

    @pl.loop(0, _ZCH, step=16)
    def _(i):
        zv[pl.ds(i, 16)] = jnp.zeros((16,), jnp.float32)

    sl = pl.ds(sid * _ZCH, _ZCH)
    pltpu.sync_copy(zv, s.at[sl])
    plsc.subcore_barrier()

    pltpu.sync_copy(i12_ref.at[wid], idx_v)
    pltpu.sync_copy(ones_v, s.at[idx_v.at[0]], add=True)
    plsc.subcore_barrier()

    pltpu.sync_copy(s.at[sl], cbuf)
    pltpu.sync_copy(cbuf, out_ref.at[pl.ds(cid * _SBUF + sid * _ZCH, _ZCH)])


def _sc_edge_counts(i12):
    return pl.kernel(
        _count_body,
        out_type=jax.ShapeDtypeStruct((_NSC * _SBUF,), jnp.float32),
        mesh=plsc.VectorSubcoreMesh(core_axis_name="c", subcore_axis_name="s"),
        scratch_types=[
            pltpu.VMEM((1, 2 * _EPW), jnp.int32),
            pltpu.VMEM((2 * _EPW,), jnp.float32),
            pltpu.VMEM((_ZCH,), jnp.float32),
            pltpu.VMEM((_ZCH,), jnp.float32),
            pltpu.VMEM_SHARED((_SBUF,), jnp.float32),
        ],
    )(i12)


def _gcn_body(cs_ref, w0_ref, b0_ref, w1_ref, b1_ref, w2_ref, b2_ref, x_ref, out_ref):
    cnt = cs_ref[0, :, 0:_N]
    cntT = cs_ref[1, :, 0:_N]
    deg_c = jnp.sum(cnt, axis=1, keepdims=True) + 1.0
    deg_r = jnp.sum(cntT, axis=0, keepdims=True) + 1.0
    eye = (
        jax.lax.broadcasted_iota(jnp.int32, (_N, _N), 0)
        == jax.lax.broadcasted_iota(jnp.int32, (_N, _N), 1)
    ).astype(jnp.float32)
    A = (cnt + eye) * jax.lax.rsqrt(deg_c) * jax.lax.rsqrt(deg_r)

    def layer(h, w_ref, b_ref, bt, din):
        w = w_ref[...]
        y = jnp.concatenate(
            [_dot(h[:, i * din : (i + 1) * din], w) for i in range(_K)], axis=1
        )
        return jnp.maximum(_dot(A, y) + bt, 0.0)

    h = jnp.concatenate(
        [x_ref[i, j] for j in range(_FPP) for i in range(_B)], axis=1
    )
    bts = [jnp.tile(b_ref[...], (1, _K)) for b_ref in (b0_ref, b1_ref, b2_ref)]
    h = layer(h, w0_ref, b0_ref, bts[0], _F)
    h = layer(h, w1_ref, b1_ref, bts[1], _SD)
    h = layer(h, w2_ref, b2_ref, bts[2], _SD)
    out_ref[0, 0, :] = jnp.mean(h, axis=0)


def _lstm_body(
    seq_ref,
    wih0f_ref,
    whh0f_ref,
    wih0b_ref,
    whh0b_ref,
    wih1f_ref,
    whh1f_ref,
    wih1b_ref,
    whh1b_ref,
    bias0_ref,
    bias1_ref,
    clsw1_ref,
    clsb1_ref,
    clsw2_ref,
    clsb2_ref,
    out_ref,
    g0_ref,
    seq1_ref,
    g1_ref,
):
    H4 = 4 * _TD

    seq = seq_ref[...]
    g0_ref[:, 0:H4] = _dot_t(seq, wih0f_ref[...]) + bias0_ref[:, 0:H4]
    g0_ref[:, H4 : 2 * H4] = _dot_t(seq, wih0b_ref[...]) + bias0_ref[:, H4 : 2 * H4]

    def cell(g, c):
        i = jax.nn.sigmoid(g[:, 0:_TD])
        f = jax.nn.sigmoid(g[:, _TD : 2 * _TD])
        gg = jnp.tanh(g[:, 2 * _TD : 3 * _TD])
        o = jax.nn.sigmoid(g[:, 3 * _TD : 4 * _TD])
        c = f * c + i * gg
        return o * jnp.tanh(c), c

    def step0(s, carry):
        hf, cf, hb, cb = carry
        gf = g0_ref[pl.ds(s * _B, _B), 0:H4] + _dot_t(hf, whh0f_ref[...])
        gb = g0_ref[pl.ds((_T - 1 - s) * _B, _B), H4 : 2 * H4] + _dot_t(
            hb, whh0b_ref[...]
        )
        hf, cf = cell(gf, cf)
        hb, cb = cell(gb, cb)
        seq1_ref[pl.ds(s * _B, _B), 0:_TD] = hf
        seq1_ref[pl.ds((_T - 1 - s) * _B, _B), _TD : 2 * _TD] = hb
        return hf, cf, hb, cb

    z = jnp.zeros((_B, _TD), jnp.float32)
    jax.lax.fori_loop(0, _T, step0, (z, z, z, z))

    seq1 = seq1_ref[...]
    g1_ref[:, 0:H4] = _dot_t(seq1, wih1f_ref[...]) + bias1_ref[:, 0:H4]
    g1_ref[:, H4 : 2 * H4] = _dot_t(seq1, wih1b_ref[...]) + bias1_ref[:, H4 : 2 * H4]

    def step1(s, carry):
        hf, cf, hb, cb = carry
        gf = g1_ref[pl.ds(s * _B, _B), 0:H4] + _dot_t(hf, whh1f_ref[...])
        gb = g1_ref[pl.ds((_T - 1 - s) * _B, _B), H4 : 2 * H4] + _dot_t(
            hb, whh1b_ref[...]
        )
        hf, cf = cell(gf, cf)
        hb, cb = cell(gb, cb)
        return hf, cf, hb, cb

    h1f, _, h1b, _ = jax.lax.fori_loop(0, _T, step1, (z, z, z, z))

    to = jnp.concatenate([h1f, h1b], axis=1)
    h = jnp.maximum(_dot(to, clsw1_ref[...]) + clsb1_ref[...], 0.0)
    out_ref[...] = _dot(h, clsw2_ref[...]) + clsb2_ref[...]


@jax.jit
def kernel(x_temporal, edge_index, gcn_W0, gcn_b0, gcn_W1, gcn_b1, gcn_W2, gcn_b2, lstm_Wih_l0f, lstm_Whh_l0f, lstm_bih_l0f, lstm_bhh_l0f, lstm_Wih_l0b, lstm_Whh_l0b, lstm_bih_l0b, lstm_bhh_l0b, lstm_Wih_l1f, lstm_Whh_l1f, lstm_bih_l1f, lstm_bhh_l1f, lstm_Wih_l1b, lstm_Whh_l1b, lstm_bih_l1b, lstm_bhh_l1b, cls_W1, cls_b1, bn_gamma, bn_beta, bn_mean, bn_var, cls_W2, cls_b2):
    f32 = jnp.float32
    H4 = 4 * _TD

    srcn = edge_index[0]
    dstn = edge_index[1]
    npad = _ESLOTS - _E
    pad = _ABUF + (jnp.arange(npad, dtype=jnp.int32) % (_ABUFP - _ABUF))
    f1 = jnp.concatenate([dstn * _STR + srcn, pad]).reshape(_NSC * _NSUB, _EPW)
    f2 = _ABUFP + jnp.concatenate([srcn * _STR + dstn, pad]).reshape(
        _NSC * _NSUB, _EPW
    )
    i12 = jnp.concatenate([f1, f2], axis=1).reshape(_NSC * _NSUB, 1, 2 * _EPW)
    parts = _sc_edge_counts(i12).reshape(_NSC, 2, _ABUFP)
    csum = parts[0] + parts[1]
    cs2 = csum[:, :_ABUF].reshape(2, _N, _STR)

    bt = [b.reshape(1, _SD) for b in (gcn_b0, gcn_b1, gcn_b2)]
    ngp = _T // _FPP
    seq = pl.pallas_call(
        _gcn_body,
        grid=(ngp,),
        in_specs=[
            pl.BlockSpec((2, _N, _STR), lambda t: (0, 0, 0)),
            pl.BlockSpec((_F, _SD), lambda t: (0, 0)),
            pl.BlockSpec((1, _SD), lambda t: (0, 0)),
            pl.BlockSpec((_SD, _SD), lambda t: (0, 0)),
            pl.BlockSpec((1, _SD), lambda t: (0, 0)),
            pl.BlockSpec((_SD, _SD), lambda t: (0, 0)),
            pl.BlockSpec((1, _SD), lambda t: (0, 0)),
            pl.BlockSpec((_B, _FPP, _N, _F), lambda t: (0, t, 0, 0)),
        ],
        out_specs=pl.BlockSpec((1, 1, _K * _SD), lambda t: (t, 0, 0)),
        out_shape=jax.ShapeDtypeStruct((ngp, 1, _K * _SD), f32),
    )(cs2, gcn_W0, bt[0], gcn_W1, bt[1], gcn_W2, bt[2], x_temporal)
    seq = seq.reshape(_TB, _SD)

    bias0 = jnp.concatenate(
        [lstm_bih_l0f + lstm_bhh_l0f, lstm_bih_l0b + lstm_bhh_l0b]
    ).reshape(1, 2 * H4)
    bias1 = jnp.concatenate(
        [lstm_bih_l1f + lstm_bhh_l1f, lstm_bih_l1b + lstm_bhh_l1b]
    ).reshape(1, 2 * H4)

    scale = bn_gamma * jax.lax.rsqrt(bn_var + 1e-5)
    w1s = cls_W1 * scale[None, :]
    b1s = ((cls_b1 - bn_mean) * scale + bn_beta).reshape(1, _CD)

    logits = pl.pallas_call(
        _lstm_body,
        out_shape=jax.ShapeDtypeStruct((_B, _NC), f32),
        scratch_shapes=[
            pltpu.VMEM((_TB, 2 * H4), f32),
            pltpu.VMEM((_TB, 2 * _TD), f32),
            pltpu.VMEM((_TB, 2 * H4), f32),
        ],
    )(
        seq,
        lstm_Wih_l0f,
        lstm_Whh_l0f,
        lstm_Wih_l0b,
        lstm_Whh_l0b,
        lstm_Wih_l1f,
        lstm_Whh_l1f,
        lstm_Wih_l1b,
        lstm_Whh_l1b,
        bias0,
        bias1,
        w1s,
        b1s,
        cls_W2,
        cls_b2.reshape(1, _NC),
    )
    return logits
